# baseline (speedup 1.0000x reference)
import jax
import jax.numpy as jnp
from jax.experimental import pallas as pl
from jax.experimental.pallas import tpu as pltpu

_COLS = 1024
_BLOCK_ROWS = 1728


def _gate_scale_kernel(idx_ref, lw_ref, feat_ref, out_ref):
    w = lw_ref[0, idx_ref[0]]
    gate = 1.0 / (1.0 + jnp.exp(-w))
    out_ref[...] = feat_ref[...] * gate


def kernel(idx, feat, layerweight):
    orig_shape = feat.shape
    total = feat.size
    rows = total // _COLS
    flat = feat.reshape(rows, _COLS)
    block_rows = _BLOCK_ROWS if rows % _BLOCK_ROWS == 0 else rows
    idx_arr = jnp.asarray(idx, dtype=jnp.int32).reshape((1,))
    out = pl.pallas_call(
        _gate_scale_kernel,
        grid=(rows // block_rows,),
        in_specs=[
            pl.BlockSpec(memory_space=pltpu.SMEM),
            pl.BlockSpec(memory_space=pltpu.SMEM),
            pl.BlockSpec((block_rows, _COLS), lambda i: (i, 0)),
        ],
        out_specs=pl.BlockSpec((block_rows, _COLS), lambda i: (i, 0)),
        out_shape=jax.ShapeDtypeStruct((rows, _COLS), jnp.float32),
    )(idx_arr, layerweight, flat)
    return out.reshape(orig_shape)


# speedup vs baseline: 1.8245x; 1.8245x over previous
import jax
import jax.numpy as jnp
from jax.experimental import pallas as pl
from jax.experimental.pallas import tpu as pltpu


def _gate_scale_kernel(idx_ref, lw_ref, feat_ref, out_ref):
    w = lw_ref[0, idx_ref[0]]
    gate = 1.0 / (1.0 + jnp.exp(-w))
    out_ref[...] = feat_ref[...] * gate


def kernel(idx, feat, layerweight):
    n0, n1, n2, n3 = feat.shape
    block = (1, n1, n2, n3)
    idx_arr = jnp.asarray(idx, dtype=jnp.int32).reshape((1,))
    return pl.pallas_call(
        _gate_scale_kernel,
        grid=(n0,),
        in_specs=[
            pl.BlockSpec(memory_space=pltpu.SMEM),
            pl.BlockSpec(memory_space=pltpu.SMEM),
            pl.BlockSpec(block, lambda i: (i, 0, 0, 0)),
        ],
        out_specs=pl.BlockSpec(block, lambda i: (i, 0, 0, 0)),
        out_shape=jax.ShapeDtypeStruct(feat.shape, feat.dtype),
    )(idx_arr, layerweight, feat)


# speedup vs baseline: 26.6599x; 14.6122x over previous
import jax
import jax.numpy as jnp
from jax.experimental import pallas as pl
from jax.experimental.pallas import tpu as pltpu


def _gate_scale_kernel(idx_ref, lw_ref, feat_ref, out_ref):
    w = lw_ref[0, idx_ref[0]]
    gate = 1.0 / (1.0 + jnp.exp(-w))
    out_ref[...] = feat_ref[...] * gate


def kernel(idx, feat, layerweight):
    n0, n1, n2, n3 = feat.shape
    feat_t = jnp.transpose(feat, (0, 2, 3, 1))
    b0 = 4 if n0 % 4 == 0 else 1
    block = (b0, n2, n3, n1)
    idx_arr = jnp.asarray(idx, dtype=jnp.int32).reshape((1,))
    out_t = pl.pallas_call(
        _gate_scale_kernel,
        grid=(n0 // b0,),
        in_specs=[
            pl.BlockSpec(memory_space=pltpu.SMEM),
            pl.BlockSpec(memory_space=pltpu.SMEM),
            pl.BlockSpec(block, lambda i: (i, 0, 0, 0)),
        ],
        out_specs=pl.BlockSpec(block, lambda i: (i, 0, 0, 0)),
        out_shape=jax.ShapeDtypeStruct((n0, n2, n3, n1), feat.dtype),
    )(idx_arr, layerweight, feat_t)
    return jnp.transpose(out_t, (0, 3, 1, 2))


# speedup vs baseline: 26.7764x; 1.0044x over previous
import jax
import jax.numpy as jnp
from jax.experimental import pallas as pl
from jax.experimental.pallas import tpu as pltpu


def _gate_scale_kernel(idx_ref, lw_ref, feat_ref, out_ref):
    w = lw_ref[0, idx_ref[0]]
    gate = 1.0 / (1.0 + jnp.exp(-w))
    out_ref[...] = feat_ref[...] * gate


def kernel(idx, feat, layerweight):
    n0, n1, n2, n3 = feat.shape
    feat_t = jnp.transpose(feat, (0, 2, 3, 1))
    b0 = 8 if n0 % 8 == 0 else 1
    block = (b0, n2, n3, n1)
    idx_arr = jnp.asarray(idx, dtype=jnp.int32).reshape((1,))
    out_t = pl.pallas_call(
        _gate_scale_kernel,
        grid=(n0 // b0,),
        in_specs=[
            pl.BlockSpec(memory_space=pltpu.SMEM),
            pl.BlockSpec(memory_space=pltpu.SMEM),
            pl.BlockSpec(block, lambda i: (i, 0, 0, 0)),
        ],
        out_specs=pl.BlockSpec(block, lambda i: (i, 0, 0, 0)),
        out_shape=jax.ShapeDtypeStruct((n0, n2, n3, n1), feat.dtype),
    )(idx_arr, layerweight, feat_t)
    return jnp.transpose(out_t, (0, 3, 1, 2))
